# baseline (speedup 1.0000x reference)
import functools

import jax
import jax.numpy as jnp
from jax import lax
from jax.experimental import pallas as pl
from jax.experimental.pallas import tpu as pltpu
from jax.experimental.pallas import tpu_sc as plsc

_NC = 2
_NS = 16
_NW = _NC * _NS

_GRP = 128
_K = 8


@functools.lru_cache(maxsize=None)
def _make_gather(V, D, R):
    rpw = R // _NW
    G = rpw // _K
    mesh = plsc.VectorSubcoreMesh(core_axis_name="c", subcore_axis_name="s")

    @functools.partial(
        pl.kernel,
        mesh=mesh,
        out_type=jax.ShapeDtypeStruct((R * _GRP, D), jnp.float32),
        scratch_types=[
            pltpu.VMEM((_K, _GRP), jnp.int32),
            pltpu.VMEM((_K * _GRP, D), jnp.float32),
            pltpu.SemaphoreType.DMA,
        ],
        compiler_params=pltpu.CompilerParams(use_tc_tiling_on_sc=False),
    )
    def k(table_hbm, idx_hbm, out_hbm, idx_v, rows_v, sem):
        wid = lax.axis_index("s") * _NC + lax.axis_index("c")
        row0 = wid * rpw

        def body(g, carry):
            r = row0 + g * _K
            pltpu.sync_copy(idx_hbm.at[pl.ds(r, _K)], idx_v)
            copies = [
                pltpu.async_copy(
                    table_hbm.at[idx_v.at[j]],
                    rows_v.at[pl.ds(j * _GRP, _GRP)],
                    sem,
                )
                for j in range(_K)
            ]
            for c in copies:
                c.wait()
            pltpu.sync_copy(rows_v, out_hbm.at[pl.ds(r * _GRP, _K * _GRP)])
            return carry

        lax.fori_loop(0, G, body, 0)

    return k


def kernel(idx_texts, embed_table):
    B, S = idx_texts.shape
    V, D = embed_table.shape
    N = B * S
    R = N // _GRP
    idx2d = idx_texts.reshape(R, _GRP)
    out = _make_gather(V, D, R)(embed_table, idx2d)
    return out.reshape(B, S, D)


# speedup vs baseline: 1.0167x; 1.0167x over previous
import functools

import jax
import jax.numpy as jnp
from jax import lax
from jax.experimental import pallas as pl
from jax.experimental.pallas import tpu as pltpu
from jax.experimental.pallas import tpu_sc as plsc

_NC = 2
_NS = 16
_NW = _NC * _NS

_C = 800
_NB = 2


@functools.lru_cache(maxsize=None)
def _make_gather(V, D, N):
    rpw = N // _NW
    G = rpw // _C
    M = G // _NB
    mesh = plsc.VectorSubcoreMesh(core_axis_name="c", subcore_axis_name="s")

    @functools.partial(
        pl.kernel,
        mesh=mesh,
        out_type=jax.ShapeDtypeStruct((N, D), jnp.float32),
        scratch_types=[
            pltpu.VMEM((rpw,), jnp.int32),
            pltpu.VMEM((_NB * _C, D), jnp.float32),
            pltpu.SemaphoreType.DMA,
            pltpu.SemaphoreType.DMA,
            pltpu.SemaphoreType.DMA,
            pltpu.SemaphoreType.DMA,
        ],
        compiler_params=pltpu.CompilerParams(use_tc_tiling_on_sc=False),
    )
    def k(table_hbm, idx_hbm, out_hbm, idx_v, rows_v, gs0, gs1, os0, os1):
        gsem = (gs0, gs1)
        osem = (os0, os1)
        wid = lax.axis_index("s") * _NC + lax.axis_index("c")
        base = wid * rpw
        pltpu.sync_copy(idx_hbm.at[pl.ds(base, rpw)], idx_v)

        def rows_slot(b):
            return rows_v.at[pl.ds(b * _C, _C)]

        def fire_gather(cur, b):
            pltpu.async_copy(
                table_hbm.at[idx_v.at[pl.ds(cur * _C, _C)]],
                rows_slot(b),
                gsem[b],
            )

        def wait_gather(b):
            pltpu.make_async_copy(
                table_hbm.at[pl.ds(0, _C)], rows_slot(b), gsem[b]
            ).wait()

        def fire_write(cur, b):
            pltpu.async_copy(
                rows_slot(b),
                out_hbm.at[pl.ds(base + cur * _C, _C)],
                osem[b],
            )

        def wait_write(b):
            pltpu.make_async_copy(
                rows_slot(b), out_hbm.at[pl.ds(base, _C)], osem[b]
            ).wait()

        for b in range(_NB):
            fire_gather(b, b)

        def body(i, carry):
            for b in range(_NB):
                cur = i * _NB + b
                wait_gather(b)
                fire_write(cur, b)
                wait_write(b)
                fire_gather(cur + _NB, b)
            return carry

        lax.fori_loop(0, M - 1, body, 0)

        for b in range(_NB):
            wait_gather(b)
            fire_write((M - 1) * _NB + b, b)
        for b in range(_NB):
            wait_write(b)

    return k


def kernel(idx_texts, embed_table):
    B, S = idx_texts.shape
    V, D = embed_table.shape
    N = B * S
    out = _make_gather(V, D, N)(embed_table, idx_texts.reshape(N))
    return out.reshape(B, S, D)
